# baseline (speedup 1.0000x reference)
import functools

import jax
import jax.numpy as jnp
from jax import lax
from jax.experimental import pallas as pl
from jax.experimental.pallas import tpu as pltpu
from jax.experimental.pallas import tpu_sc as plsc

_D = 32
_BATCH = 16384
_FIELDS = 26
_NC = 2
_NS = 16
_NW = _NC * _NS
_BPW = _BATCH // _NW
_NB = 16
_NSLAB = _BPW // _NB
_NBUF = 4
_LAG = 2


def _make_lookup():
    mesh = plsc.VectorSubcoreMesh(core_axis_name="c", subcore_axis_name="s")

    @functools.partial(
        pl.kernel,
        out_type=jax.ShapeDtypeStruct((_BATCH, _FIELDS, _D), jnp.float32),
        mesh=mesh,
        scratch_types=[
            pltpu.VMEM((_BPW, _FIELDS), jnp.int32),
            pltpu.VMEM((_NBUF, _NB, _FIELDS, _D), jnp.float32),
            pltpu.SemaphoreType.DMA,
            pltpu.SemaphoreType.DMA,
        ],
        compiler_params=pltpu.CompilerParams(use_tc_tiling_on_sc=False),
    )
    def lookup(idx_hbm, table_hbm, out_hbm, idx_v, rows_v, sem_g, sem_s):
        wid = lax.axis_index("s") * _NC + lax.axis_index("c")
        bbase = wid * _BPW
        pltpu.sync_copy(idx_hbm.at[pl.ds(bbase, _BPW)], idx_v)

        def fire_gathers(j):
            buf = lax.rem(j, _NBUF)
            for i in range(_NB):
                pltpu.async_copy(
                    table_hbm.at[idx_v.at[j * _NB + i]], rows_v.at[buf, i],
                    sem_g)

        def fire_store(j):
            pltpu.async_copy(
                rows_v.at[lax.rem(j, _NBUF)],
                out_hbm.at[pl.ds(bbase + j * _NB, _NB)], sem_s)

        def wait_gathers():
            pltpu.make_async_copy(
                out_hbm.at[pl.ds(bbase, _NB)], rows_v.at[0], sem_g).wait()

        def wait_store():
            pltpu.make_async_copy(
                out_hbm.at[pl.ds(bbase, _NB)], rows_v.at[0], sem_s).wait()

        for j in range(_LAG):
            fire_gathers(j)

        def step(j, carry):
            wait_gathers()
            fire_store(j)
            @pl.when(j >= _LAG)
            def _():
                wait_store()
            @pl.when(j + _LAG < _NSLAB)
            def _():
                fire_gathers(j + _LAG)
            return carry

        lax.fori_loop(0, _NSLAB, step, 0)

        for _ in range(_LAG):
            wait_store()

    return lookup


_lookup = _make_lookup()


def kernel(input, embedding):
    return _lookup(input.astype(jnp.int32), embedding)


# speedup vs baseline: 1.1442x; 1.1442x over previous
import functools

import jax
import jax.numpy as jnp
from jax import lax
from jax.experimental import pallas as pl
from jax.experimental.pallas import tpu as pltpu
from jax.experimental.pallas import tpu_sc as plsc

_D = 32
_BATCH = 16384
_FIELDS = 26
_VOCAB = 1000000
_NC = 2
_NS = 16
_NW = _NC * _NS
_L = 16

_VW = 512
_NWIN = _VOCAB // _VW
_WPW = 1952 // _NW
_RTAIL = _NWIN * (_VW // 4)

_BPW = _BATCH // _NW
_NB = 32
_NCH = _BPW // _NB


def _make_conv():
    mesh = plsc.VectorSubcoreMesh(core_axis_name="c", subcore_axis_name="s")

    @functools.partial(
        pl.kernel,
        out_type=jax.ShapeDtypeStruct((_VOCAB // 4, 128), jnp.float32),
        mesh=mesh,
        scratch_types=[
            pltpu.VMEM((_D, _VW), jnp.float32),
            pltpu.VMEM((_D, _VW), jnp.float32),
            pltpu.VMEM((_VW // 4, 128), jnp.float32),
            pltpu.VMEM((_VW // 4, 128), jnp.float32),
            pltpu.SemaphoreType.DMA,
            pltpu.SemaphoreType.DMA,
        ],
        compiler_params=pltpu.CompilerParams(
            use_tc_tiling_on_sc=True, needs_layout_passes=False),
    )
    def conv(tab_t_hbm, tail_hbm, cvt_hbm, in0, in1, out0, out1,
             sem_r, sem_w):
        wid = lax.axis_index("s") * _NC + lax.axis_index("c")
        in_bufs = (in0, in1)
        out_bufs = (out0, out1)
        nwin = _WPW + jnp.where(wid == 0, 1, 0)

        def win_of(k):
            return jnp.where(k == _WPW, 1952, wid * _WPW + k)

        def fire_read(k, buf):
            v0 = pl.multiple_of(win_of(k) * _VW, _VW)
            pltpu.async_copy(tab_t_hbm.at[:, pl.ds(v0, _VW)], buf, sem_r)

        def fire_write(k, buf):
            r0 = pl.multiple_of(win_of(k) * (_VW // 4), _VW // 4)
            pltpu.async_copy(buf, cvt_hbm.at[pl.ds(r0, _VW // 4)], sem_w)

        def wait_read():
            pltpu.make_async_copy(
                tab_t_hbm.at[:, pl.ds(0, _VW)], in0, sem_r).wait()

        def wait_write():
            pltpu.make_async_copy(
                cvt_hbm.at[pl.ds(0, _VW // 4)], out0, sem_w).wait()

        lane = lax.iota(jnp.int32, _L)
        idx_row = [lane + 16 * h for h in range(2)]
        zero16 = jnp.full((_L,), 0, jnp.int32)

        def transpose_window(src, dst):
            @plsc.parallel_loop(0, _VW // 4, 1)
            def per_rr(rr):
                base = rr * 4
                for jg in range(8):
                    idx_col = zero16 + (base + jg // 2)
                    vec = plsc.load_gather(src, [idx_row[jg % 2], idx_col])
                    dst[rr, pl.ds(16 * jg, _L)] = vec

        @pl.when(wid == 1)
        def _():
            pltpu.sync_copy(tail_hbm, in0.at[pl.ds(0, 16), pl.ds(0, 128)])
            pltpu.sync_copy(
                in0.at[pl.ds(0, 16), pl.ds(0, 128)],
                cvt_hbm.at[pl.ds(pl.multiple_of(_RTAIL, 8), 16)])

        fire_read(0, in0)

        def step(j, carry):
            for half in range(2):
                k = 2 * j + half

                @pl.when(k < nwin)
                def _():
                    src = in_bufs[half]
                    dst = out_bufs[half]
                    wait_read()

                    @pl.when(k + 1 < nwin)
                    def _():
                        fire_read(k + 1, in_bufs[1 - half])

                    @pl.when(k >= 2)
                    def _():
                        wait_write()

                    transpose_window(src, dst)
                    fire_write(k, dst)
            return carry

        lax.fori_loop(0, (_WPW + 2) // 2, step, 0)

        @pl.when(nwin >= 2)
        def _():
            wait_write()
        wait_write()

    return conv


def _make_lookup():
    mesh = plsc.VectorSubcoreMesh(core_axis_name="c", subcore_axis_name="s")

    @functools.partial(
        pl.kernel,
        out_type=jax.ShapeDtypeStruct((_FIELDS, _D, _BATCH), jnp.float32),
        mesh=mesh,
        scratch_types=[
            pltpu.VMEM((_BPW, _FIELDS), jnp.int32),
            pltpu.VMEM((_NB, _FIELDS, _D), jnp.float32),
            pltpu.VMEM((_NB, _FIELDS, _D), jnp.float32),
            pltpu.VMEM((_FIELDS, _D, _NB), jnp.float32),
            pltpu.VMEM((_FIELDS, _D, _NB), jnp.float32),
            pltpu.SemaphoreType.DMA,
            pltpu.SemaphoreType.DMA,
        ],
        compiler_params=pltpu.CompilerParams(
            use_tc_tiling_on_sc=False, needs_layout_passes=False),
    )
    def lookup(idx_hbm, table_hbm, out_hbm, idx_v, rows0, rows1, tout0,
               tout1, sem_g, sem_s):
        wid = lax.axis_index("s") * _NC + lax.axis_index("c")
        bbase = wid * _BPW
        pltpu.sync_copy(idx_hbm.at[pl.ds(bbase, _BPW)], idx_v)

        rows_bufs = (rows0, rows1)
        tout_bufs = (tout0, tout1)

        def fire_gathers(k, rows):
            for i in range(_NB):
                pltpu.async_copy(
                    table_hbm.at[idx_v.at[k * _NB + i]], rows.at[i], sem_g)

        def fire_store(k, tout):
            pltpu.async_copy(
                tout, out_hbm.at[:, :, pl.ds(bbase + k * _NB, _NB)], sem_s)

        def wait_gathers():
            pltpu.make_async_copy(
                out_hbm.at[:, :, pl.ds(bbase, _NB)], rows0, sem_g).wait()

        def wait_store():
            pltpu.make_async_copy(
                out_hbm.at[:, :, pl.ds(bbase, _NB)], tout0, sem_s).wait()

        lane = lax.iota(jnp.int32, _L)
        idx_b = [lane + g * _L for g in range(_NB // _L)]
        zero16 = jnp.full((_L,), 0, jnp.int32)

        def transpose_chunk(rows, tout):
            @plsc.parallel_loop(0, _FIELDS, 1)
            def per_f(f):
                idx_f = zero16 + f
                for e in range(_D):
                    idx_e = jnp.full((_L,), e, jnp.int32)
                    for g in range(_NB // _L):
                        vec = plsc.load_gather(rows, [idx_b[g], idx_f, idx_e])
                        tout[f, e, pl.ds(g * _L, _L)] = vec

        fire_gathers(0, rows0)
        fire_gathers(1, rows1)

        def step(j, carry):
            for half in range(2):
                k = 2 * j + half
                rows = rows_bufs[half]
                tout = tout_bufs[half]
                wait_gathers()

                @pl.when(k >= 2)
                def _():
                    wait_store()

                transpose_chunk(rows, tout)
                fire_store(k, tout)

                @pl.when(k + 2 < _NCH)
                def _():
                    fire_gathers(k + 2, rows)
            return carry

        lax.fori_loop(0, _NCH // 2, step, 0)

        wait_store()
        wait_store()

    return lookup


_conv = _make_conv()
_lookup = _make_lookup()


def kernel(input, embedding):
    tail_rm = embedding[_NWIN * _VW:].reshape(16, 128)
    cvt = _conv(embedding.T, tail_rm)
    table_rm = cvt.reshape(_VOCAB, _D)
    out_t = _lookup(input.astype(jnp.int32), table_rm)
    return jnp.transpose(out_t, (2, 0, 1))
